# baseline (speedup 1.0000x reference)
import functools

import jax
import jax.numpy as jnp
from jax import lax
from jax.experimental import pallas as pl
from jax.experimental.pallas import tpu as pltpu
from jax.experimental.pallas import tpu_sc as plsc

ROWS = 64
VOCAB = 1_000_000
VBLK = 8192
NBLK = (VOCAB + VBLK - 1) // VBLK
FLAT = ROWS * VOCAB
CHUNK = ROWS * VBLK


def _fused_body(x_ref, tok_ref, z_ref, vmax_ref, vidx_ref):
    i = pl.program_id(0)
    nb = pl.num_programs(0)
    x = x_ref[...]

    @pl.when(i == 0)
    def _init():
        vmax_ref[...] = jnp.full((ROWS, 1), -jnp.inf, jnp.float32)
        vidx_ref[...] = jnp.zeros((ROWS, 1), jnp.int32)

    z_ref[...] = jnp.zeros((ROWS, VBLK), jnp.float32)

    bmax = jnp.max(x, axis=1, keepdims=True)

    @pl.when(jnp.logical_and(i < nb - 1, jnp.any(bmax > vmax_ref[...])))
    def _update():
        upd = bmax > vmax_ref[...]
        col = lax.broadcasted_iota(jnp.int32, (ROWS, VBLK), 1) + i * VBLK
        bidx = jnp.min(
            jnp.where(x == bmax, col, jnp.int32(2**31 - 1)),
            axis=1, keepdims=True,
        )
        vidx_ref[...] = jnp.where(upd, bidx, vidx_ref[...])
        vmax_ref[...] = jnp.where(upd, bmax, vmax_ref[...])

    @pl.when(i == nb - 1)
    def _tail():
        col = lax.broadcasted_iota(jnp.int32, (ROWS, VBLK), 1) + i * VBLK
        xm = jnp.where(col < VOCAB, x, -jnp.inf)
        tmax = jnp.max(xm, axis=1, keepdims=True)
        upd = tmax > vmax_ref[...]
        bidx = jnp.min(
            jnp.where(xm == tmax, col, jnp.int32(2**31 - 1)),
            axis=1, keepdims=True,
        )
        vidx_ref[...] = jnp.where(upd, bidx, vidx_ref[...])
        tok_ref[...] = vidx_ref[...]


def _fused_pass(logits):
    return pl.pallas_call(
        _fused_body,
        grid=(NBLK,),
        in_specs=[pl.BlockSpec((ROWS, VBLK), lambda i: (0, i))],
        out_specs=[
            pl.BlockSpec((ROWS, 1), lambda i: (0, 0)),
            pl.BlockSpec((ROWS, VBLK), lambda i: (0, i)),
        ],
        out_shape=[
            jax.ShapeDtypeStruct((ROWS, 1), jnp.int32),
            jax.ShapeDtypeStruct((ROWS, VOCAB), jnp.float32),
        ],
        scratch_shapes=[
            pltpu.VMEM((ROWS, 1), jnp.float32),
            pltpu.VMEM((ROWS, 1), jnp.int32),
        ],
        compiler_params=pltpu.CompilerParams(
            dimension_semantics=("arbitrary",)
        ),
    )(logits)


SUB = 512


def _ones_body(tok_sref, blk_ref, out_ref):
    r = pl.program_id(0)
    g = r // 8
    base = (tok_sref[r] // SUB) * SUB
    col = lax.broadcasted_iota(jnp.int32, (8, SUB), 1) + base
    rowi = lax.broadcasted_iota(jnp.int32, (8, SUB), 0)
    acc = jnp.zeros((8, SUB), jnp.float32)
    for i in range(8):
        t_i = tok_sref[g * 8 + i]
        acc = jnp.where(jnp.logical_and(rowi == i, col == t_i), 1.0, acc)
    out_ref[...] = acc


def _scatter_ones(tok, probs):
    return pl.pallas_call(
        _ones_body,
        grid_spec=pltpu.PrefetchScalarGridSpec(
            num_scalar_prefetch=1,
            grid=(ROWS,),
            in_specs=[
                pl.BlockSpec(
                    (8, SUB), lambda r, tok_ref: (r // 8, tok_ref[r] // SUB)
                ),
            ],
            out_specs=pl.BlockSpec(
                (8, SUB), lambda r, tok_ref: (r // 8, tok_ref[r] // SUB)
            ),
        ),
        out_shape=jax.ShapeDtypeStruct((ROWS, VOCAB), jnp.float32),
        input_output_aliases={1: 0},
        compiler_params=pltpu.CompilerParams(
            dimension_semantics=("arbitrary",)
        ),
    )(tok, probs)


def kernel(logits, eos_token_ids):
    tok2, probs2d = _fused_pass(logits)
    tokens = tok2.reshape(ROWS)
    probs = _scatter_ones(tokens, probs2d)
    return tokens, probs


# speedup vs baseline: 1.1554x; 1.1554x over previous
import functools

import jax
import jax.numpy as jnp
from jax import lax
from jax.experimental import pallas as pl
from jax.experimental.pallas import tpu as pltpu
from jax.experimental.pallas import tpu_sc as plsc

ROWS = 64
VOCAB = 1_000_000
VBLK = 16384
NBLK = (VOCAB + VBLK - 1) // VBLK
FLAT = ROWS * VOCAB
CHUNK = ROWS * VBLK


def _fused_body(x_ref, tok_ref, z_ref, vmax_ref, vidx_ref):
    i = pl.program_id(0)
    nb = pl.num_programs(0)
    x = x_ref[...]

    @pl.when(i == 0)
    def _init():
        vmax_ref[...] = jnp.full((ROWS, 1), -jnp.inf, jnp.float32)
        vidx_ref[...] = jnp.zeros((ROWS, 1), jnp.int32)

    z_ref[...] = jnp.zeros((ROWS, VBLK), jnp.float32)

    bmax = jnp.max(x, axis=1, keepdims=True)

    @pl.when(jnp.logical_and(i < nb - 1, jnp.any(bmax > vmax_ref[...])))
    def _update():
        upd = bmax > vmax_ref[...]
        col = lax.broadcasted_iota(jnp.int32, (ROWS, VBLK), 1) + i * VBLK
        bidx = jnp.min(
            jnp.where(x == bmax, col, jnp.int32(2**31 - 1)),
            axis=1, keepdims=True,
        )
        vidx_ref[...] = jnp.where(upd, bidx, vidx_ref[...])
        vmax_ref[...] = jnp.where(upd, bmax, vmax_ref[...])

    @pl.when(i == nb - 1)
    def _tail():
        col = lax.broadcasted_iota(jnp.int32, (ROWS, VBLK), 1) + i * VBLK
        xm = jnp.where(col < VOCAB, x, -jnp.inf)
        tmax = jnp.max(xm, axis=1, keepdims=True)
        upd = tmax > vmax_ref[...]
        bidx = jnp.min(
            jnp.where(xm == tmax, col, jnp.int32(2**31 - 1)),
            axis=1, keepdims=True,
        )
        vidx_ref[...] = jnp.where(upd, bidx, vidx_ref[...])
        tok_ref[...] = vidx_ref[...]


def _fused_pass(logits):
    return pl.pallas_call(
        _fused_body,
        grid=(NBLK,),
        in_specs=[pl.BlockSpec((ROWS, VBLK), lambda i: (0, i))],
        out_specs=[
            pl.BlockSpec((ROWS, 1), lambda i: (0, 0)),
            pl.BlockSpec((ROWS, VBLK), lambda i: (0, i)),
        ],
        out_shape=[
            jax.ShapeDtypeStruct((ROWS, 1), jnp.int32),
            jax.ShapeDtypeStruct((ROWS, VOCAB), jnp.float32),
        ],
        scratch_shapes=[
            pltpu.VMEM((ROWS, 1), jnp.float32),
            pltpu.VMEM((ROWS, 1), jnp.int32),
        ],
        compiler_params=pltpu.CompilerParams(
            dimension_semantics=("arbitrary",)
        ),
    )(logits)


SUB = 512


def _ones_body(tok_sref, blk_ref, out_ref):
    r = pl.program_id(0)
    g = r // 8
    base = (tok_sref[r] // SUB) * SUB
    col = lax.broadcasted_iota(jnp.int32, (8, SUB), 1) + base
    rowi = lax.broadcasted_iota(jnp.int32, (8, SUB), 0)
    acc = jnp.zeros((8, SUB), jnp.float32)
    for i in range(8):
        t_i = tok_sref[g * 8 + i]
        acc = jnp.where(jnp.logical_and(rowi == i, col == t_i), 1.0, acc)
    out_ref[...] = acc


def _scatter_ones(tok, probs):
    return pl.pallas_call(
        _ones_body,
        grid_spec=pltpu.PrefetchScalarGridSpec(
            num_scalar_prefetch=1,
            grid=(ROWS,),
            in_specs=[
                pl.BlockSpec(
                    (8, SUB), lambda r, tok_ref: (r // 8, tok_ref[r] // SUB)
                ),
            ],
            out_specs=pl.BlockSpec(
                (8, SUB), lambda r, tok_ref: (r // 8, tok_ref[r] // SUB)
            ),
        ),
        out_shape=jax.ShapeDtypeStruct((ROWS, VOCAB), jnp.float32),
        input_output_aliases={1: 0},
        compiler_params=pltpu.CompilerParams(
            dimension_semantics=("arbitrary",)
        ),
    )(tok, probs)


def kernel(logits, eos_token_ids):
    tok2, probs2d = _fused_pass(logits)
    tokens = tok2.reshape(ROWS)
    probs = _scatter_ones(tokens, probs2d)
    return tokens, probs


# speedup vs baseline: 1.2274x; 1.0623x over previous
import functools

import jax
import jax.numpy as jnp
from jax import lax
from jax.experimental import pallas as pl
from jax.experimental.pallas import tpu as pltpu
from jax.experimental.pallas import tpu_sc as plsc

ROWS = 64
VOCAB = 1_000_000
VBLK = 32768
NBLK = (VOCAB + VBLK - 1) // VBLK
FLAT = ROWS * VOCAB
CHUNK = ROWS * VBLK


def _fused_body(x_ref, tok_ref, z_ref, vmax_ref, vidx_ref):
    i = pl.program_id(0)
    nb = pl.num_programs(0)
    x = x_ref[...]

    @pl.when(i == 0)
    def _init():
        vmax_ref[...] = jnp.full((ROWS, 1), -jnp.inf, jnp.float32)
        vidx_ref[...] = jnp.zeros((ROWS, 1), jnp.int32)

    z_ref[...] = jnp.zeros((ROWS, VBLK), jnp.float32)

    bmax = jnp.max(x, axis=1, keepdims=True)

    @pl.when(jnp.logical_and(i < nb - 1, jnp.any(bmax > vmax_ref[...])))
    def _update():
        upd = bmax > vmax_ref[...]
        col = lax.broadcasted_iota(jnp.int32, (ROWS, VBLK), 1) + i * VBLK
        bidx = jnp.min(
            jnp.where(x == bmax, col, jnp.int32(2**31 - 1)),
            axis=1, keepdims=True,
        )
        vidx_ref[...] = jnp.where(upd, bidx, vidx_ref[...])
        vmax_ref[...] = jnp.where(upd, bmax, vmax_ref[...])

    @pl.when(i == nb - 1)
    def _tail():
        col = lax.broadcasted_iota(jnp.int32, (ROWS, VBLK), 1) + i * VBLK
        xm = jnp.where(col < VOCAB, x, -jnp.inf)
        tmax = jnp.max(xm, axis=1, keepdims=True)
        upd = tmax > vmax_ref[...]
        bidx = jnp.min(
            jnp.where(xm == tmax, col, jnp.int32(2**31 - 1)),
            axis=1, keepdims=True,
        )
        vidx_ref[...] = jnp.where(upd, bidx, vidx_ref[...])
        tok_ref[...] = vidx_ref[...]


def _fused_pass(logits):
    return pl.pallas_call(
        _fused_body,
        grid=(NBLK,),
        in_specs=[pl.BlockSpec((ROWS, VBLK), lambda i: (0, i))],
        out_specs=[
            pl.BlockSpec((ROWS, 1), lambda i: (0, 0)),
            pl.BlockSpec((ROWS, VBLK), lambda i: (0, i)),
        ],
        out_shape=[
            jax.ShapeDtypeStruct((ROWS, 1), jnp.int32),
            jax.ShapeDtypeStruct((ROWS, VOCAB), jnp.float32),
        ],
        scratch_shapes=[
            pltpu.VMEM((ROWS, 1), jnp.float32),
            pltpu.VMEM((ROWS, 1), jnp.int32),
        ],
        compiler_params=pltpu.CompilerParams(
            dimension_semantics=("arbitrary",)
        ),
    )(logits)


SUB = 512


def _ones_body(tok_sref, blk_ref, out_ref):
    r = pl.program_id(0)
    g = r // 8
    base = (tok_sref[r] // SUB) * SUB
    col = lax.broadcasted_iota(jnp.int32, (8, SUB), 1) + base
    rowi = lax.broadcasted_iota(jnp.int32, (8, SUB), 0)
    acc = jnp.zeros((8, SUB), jnp.float32)
    for i in range(8):
        t_i = tok_sref[g * 8 + i]
        acc = jnp.where(jnp.logical_and(rowi == i, col == t_i), 1.0, acc)
    out_ref[...] = acc


def _scatter_ones(tok, probs):
    return pl.pallas_call(
        _ones_body,
        grid_spec=pltpu.PrefetchScalarGridSpec(
            num_scalar_prefetch=1,
            grid=(ROWS,),
            in_specs=[
                pl.BlockSpec(
                    (8, SUB), lambda r, tok_ref: (r // 8, tok_ref[r] // SUB)
                ),
            ],
            out_specs=pl.BlockSpec(
                (8, SUB), lambda r, tok_ref: (r // 8, tok_ref[r] // SUB)
            ),
        ),
        out_shape=jax.ShapeDtypeStruct((ROWS, VOCAB), jnp.float32),
        input_output_aliases={1: 0},
        compiler_params=pltpu.CompilerParams(
            dimension_semantics=("arbitrary",)
        ),
    )(tok, probs)


def kernel(logits, eos_token_ids):
    tok2, probs2d = _fused_pass(logits)
    tokens = tok2.reshape(ROWS)
    probs = _scatter_ones(tokens, probs2d)
    return tokens, probs


# speedup vs baseline: 1.4455x; 1.1777x over previous
import functools

import jax
import jax.numpy as jnp
from jax import lax
from jax.experimental import pallas as pl
from jax.experimental.pallas import tpu as pltpu
from jax.experimental.pallas import tpu_sc as plsc

ROWS = 64
VOCAB = 1_000_000
VBLK = 32768
NBLK = (VOCAB + VBLK - 1) // VBLK
FLAT = ROWS * VOCAB
CHUNK = ROWS * VBLK


def _fused_body(x_ref, tok_ref, z_ref, vmax_ref, vidx_ref):
    i = pl.program_id(0)
    nb = pl.num_programs(0)
    x = x_ref[...]

    @pl.when(i == 0)
    def _init():
        vmax_ref[...] = jnp.full((ROWS, 1), -jnp.inf, jnp.float32)
        vidx_ref[...] = jnp.zeros((ROWS, 1), jnp.int32)

    z_ref[...] = jnp.zeros((ROWS, VBLK), jnp.float32)

    bmax = jnp.max(x, axis=1, keepdims=True)

    @pl.when(jnp.logical_and(i < nb - 1, jnp.any(bmax > vmax_ref[...])))
    def _update():
        upd = bmax > vmax_ref[...]
        col = lax.broadcasted_iota(jnp.int32, (ROWS, VBLK), 1) + i * VBLK
        bidx = jnp.min(
            jnp.where(x == bmax, col, jnp.int32(2**31 - 1)),
            axis=1, keepdims=True,
        )
        vidx_ref[...] = jnp.where(upd, bidx, vidx_ref[...])
        vmax_ref[...] = jnp.where(upd, bmax, vmax_ref[...])

    @pl.when(i == nb - 1)
    def _tail():
        col = lax.broadcasted_iota(jnp.int32, (ROWS, VBLK), 1) + i * VBLK
        xm = jnp.where(col < VOCAB, x, -jnp.inf)
        tmax = jnp.max(xm, axis=1, keepdims=True)
        upd = tmax > vmax_ref[...]
        bidx = jnp.min(
            jnp.where(xm == tmax, col, jnp.int32(2**31 - 1)),
            axis=1, keepdims=True,
        )
        vidx_ref[...] = jnp.where(upd, bidx, vidx_ref[...])
        tok_ref[...] = vidx_ref[...]


def _fused_pass(logits):
    return pl.pallas_call(
        _fused_body,
        grid=(NBLK,),
        in_specs=[pl.BlockSpec((ROWS, VBLK), lambda i: (0, i))],
        out_specs=[
            pl.BlockSpec((ROWS, 1), lambda i: (0, 0)),
            pl.BlockSpec((ROWS, VBLK), lambda i: (0, i)),
        ],
        out_shape=[
            jax.ShapeDtypeStruct((ROWS, 1), jnp.int32),
            jax.ShapeDtypeStruct((ROWS, VOCAB), jnp.float32),
        ],
        scratch_shapes=[
            pltpu.VMEM((ROWS, 1), jnp.float32),
            pltpu.VMEM((ROWS, 1), jnp.int32),
        ],
        compiler_params=pltpu.CompilerParams(
            dimension_semantics=("arbitrary",)
        ),
    )(logits)


SUB = 128


def _ones_body(tok_sref, tokv_ref, probs_ref, out_ref, oh_ref, sem):
    lane = lax.broadcasted_iota(jnp.int32, (8, SUB), 1)
    for w in range(ROWS):
        g = w // 8
        base_w = (tok_sref[w] // SUB) * SUB
        tok_g = tokv_ref[pl.ds(g * 8, 8), :]
        oh_ref[w] = ((tok_g - base_w) == lane).astype(jnp.float32)
    for w in range(ROWS):
        g = w // 8
        base_w = (tok_sref[w] // SUB) * SUB
        pltpu.make_async_copy(
            oh_ref.at[w],
            out_ref.at[pl.ds(g * 8, 8), pl.ds(base_w, SUB)],
            sem,
        ).start()
    for w in range(ROWS):
        g = w // 8
        base_w = (tok_sref[w] // SUB) * SUB
        pltpu.make_async_copy(
            oh_ref.at[w],
            out_ref.at[pl.ds(g * 8, 8), pl.ds(base_w, SUB)],
            sem,
        ).wait()


def _scatter_ones(tok, tok2, probs):
    return pl.pallas_call(
        _ones_body,
        grid_spec=pltpu.PrefetchScalarGridSpec(
            num_scalar_prefetch=1,
            grid=(1,),
            in_specs=[
                pl.BlockSpec((ROWS, 1), lambda i, tok_ref: (0, 0)),
                pl.BlockSpec(memory_space=pl.ANY),
            ],
            out_specs=pl.BlockSpec(memory_space=pl.ANY),
            scratch_shapes=[
                pltpu.VMEM((ROWS, 8, SUB), jnp.float32),
                pltpu.SemaphoreType.DMA,
            ],
        ),
        out_shape=jax.ShapeDtypeStruct((ROWS, VOCAB), jnp.float32),
        input_output_aliases={2: 0},
        compiler_params=pltpu.CompilerParams(
            dimension_semantics=("arbitrary",)
        ),
    )(tok, tok2, probs)


def kernel(logits, eos_token_ids):
    tok2, probs2d = _fused_pass(logits)
    tokens = tok2.reshape(ROWS)
    probs = _scatter_ones(tokens, tok2, probs2d)
    return tokens, probs


# speedup vs baseline: 1.4496x; 1.0029x over previous
import functools

import jax
import jax.numpy as jnp
from jax import lax
from jax.experimental import pallas as pl
from jax.experimental.pallas import tpu as pltpu
from jax.experimental.pallas import tpu_sc as plsc

ROWS = 64
VOCAB = 1_000_000
VBLK = 32768
NBLK = (VOCAB + VBLK - 1) // VBLK
FLAT = ROWS * VOCAB
CHUNK = ROWS * VBLK


def _fused_body(x_ref, tok_ref, z_ref, vmax_ref, vblk_ref, best_ref):
    i = pl.program_id(0)
    nb = pl.num_programs(0)
    x = x_ref[...]

    @pl.when(i == 0)
    def _init():
        vmax_ref[...] = jnp.full((ROWS, 1), -jnp.inf, jnp.float32)
        vblk_ref[...] = jnp.zeros((ROWS, 1), jnp.int32)

    z_ref[...] = jnp.zeros((ROWS, VBLK), jnp.float32)

    @pl.when(i < nb - 1)
    def _main():
        bmax = jnp.max(x, axis=1, keepdims=True)
        upd = bmax > vmax_ref[...]
        vmax_ref[...] = jnp.where(upd, bmax, vmax_ref[...])
        vblk_ref[...] = jnp.where(upd, i, vblk_ref[...])
        best_ref[...] = jnp.where(upd, x, best_ref[...])

    @pl.when(i == nb - 1)
    def _tail():
        col = lax.broadcasted_iota(jnp.int32, (ROWS, VBLK), 1)
        xm = jnp.where(col + i * VBLK < VOCAB, x, -jnp.inf)
        bmax = jnp.max(xm, axis=1, keepdims=True)
        upd = bmax > vmax_ref[...]
        vmax_ref[...] = jnp.where(upd, bmax, vmax_ref[...])
        vblk_ref[...] = jnp.where(upd, i, vblk_ref[...])
        best_ref[...] = jnp.where(upd, xm, best_ref[...])
        best = best_ref[...]
        gmax = vmax_ref[...]
        bidx = jnp.min(
            jnp.where(best == gmax, col, jnp.int32(2**31 - 1)),
            axis=1, keepdims=True,
        )
        tok_ref[...] = vblk_ref[...] * VBLK + bidx


def _fused_pass(logits):
    return pl.pallas_call(
        _fused_body,
        grid=(NBLK,),
        in_specs=[pl.BlockSpec((ROWS, VBLK), lambda i: (0, i))],
        out_specs=[
            pl.BlockSpec((ROWS, 1), lambda i: (0, 0)),
            pl.BlockSpec((ROWS, VBLK), lambda i: (0, i)),
        ],
        out_shape=[
            jax.ShapeDtypeStruct((ROWS, 1), jnp.int32),
            jax.ShapeDtypeStruct((ROWS, VOCAB), jnp.float32),
        ],
        scratch_shapes=[
            pltpu.VMEM((ROWS, 1), jnp.float32),
            pltpu.VMEM((ROWS, 1), jnp.int32),
            pltpu.VMEM((ROWS, VBLK), jnp.float32),
        ],
        compiler_params=pltpu.CompilerParams(
            dimension_semantics=("arbitrary",)
        ),
    )(logits)


SUB = 128


def _ones_body(tok_sref, tokv_ref, probs_ref, out_ref, oh_ref, sem):
    lane = lax.broadcasted_iota(jnp.int32, (8, SUB), 1)
    for w in range(ROWS):
        g = w // 8
        base_w = (tok_sref[w] // SUB) * SUB
        tok_g = tokv_ref[pl.ds(g * 8, 8), :]
        oh_ref[w] = ((tok_g - base_w) == lane).astype(jnp.float32)
    for w in range(ROWS):
        g = w // 8
        base_w = (tok_sref[w] // SUB) * SUB
        pltpu.make_async_copy(
            oh_ref.at[w],
            out_ref.at[pl.ds(g * 8, 8), pl.ds(base_w, SUB)],
            sem,
        ).start()
    for w in range(ROWS):
        g = w // 8
        base_w = (tok_sref[w] // SUB) * SUB
        pltpu.make_async_copy(
            oh_ref.at[w],
            out_ref.at[pl.ds(g * 8, 8), pl.ds(base_w, SUB)],
            sem,
        ).wait()


def _scatter_ones(tok, tok2, probs):
    return pl.pallas_call(
        _ones_body,
        grid_spec=pltpu.PrefetchScalarGridSpec(
            num_scalar_prefetch=1,
            grid=(1,),
            in_specs=[
                pl.BlockSpec((ROWS, 1), lambda i, tok_ref: (0, 0)),
                pl.BlockSpec(memory_space=pl.ANY),
            ],
            out_specs=pl.BlockSpec(memory_space=pl.ANY),
            scratch_shapes=[
                pltpu.VMEM((ROWS, 8, SUB), jnp.float32),
                pltpu.SemaphoreType.DMA,
            ],
        ),
        out_shape=jax.ShapeDtypeStruct((ROWS, VOCAB), jnp.float32),
        input_output_aliases={2: 0},
        compiler_params=pltpu.CompilerParams(
            dimension_semantics=("arbitrary",)
        ),
    )(tok, tok2, probs)


def kernel(logits, eos_token_ids):
    tok2, probs2d = _fused_pass(logits)
    tokens = tok2.reshape(ROWS)
    probs = _scatter_ones(tokens, tok2, probs2d)
    return tokens, probs
